# baseline (speedup 1.0000x reference)
import functools

import jax
import jax.numpy as jnp
from jax import lax
from jax.experimental import pallas as pl
from jax.experimental.pallas import tpu as pltpu
from jax.experimental.pallas import tpu_sc as plsc

_NUM_CORES = 2
_NUM_SUBCORES = 16
_NUM_WORKERS = _NUM_CORES * _NUM_SUBCORES
_LANES = 16
_IDX_CHUNK = 128


@functools.partial(jax.jit, static_argnames=())
def kernel(indices, items):
    B = indices.shape[0]
    V, D = items.shape
    b_per_w = B // _NUM_WORKERS
    n_chunks = b_per_w // _IDX_CHUNK

    mesh = plsc.VectorSubcoreMesh(core_axis_name="c", subcore_axis_name="s")

    @functools.partial(
        pl.kernel,
        mesh=mesh,
        compiler_params=pltpu.CompilerParams(use_tc_tiling_on_sc=False),
        out_type=jax.ShapeDtypeStruct((B, D), jnp.float32),
        scratch_types=[
            pltpu.VMEM((b_per_w,), jnp.float32),
            pltpu.VMEM((n_chunks, _IDX_CHUNK), jnp.int32),
            pltpu.VMEM((b_per_w, D), jnp.float32),
            pltpu.SemaphoreType.DMA,
        ],
    )
    def gather_kernel(ind_hbm, items_hbm, out_hbm, ind_v, idx_v, rows_v, sem):
        wid = lax.axis_index("s") * _NUM_CORES + lax.axis_index("c")
        base = wid * b_per_w
        pltpu.sync_copy(ind_hbm.at[pl.ds(base, b_per_w)], ind_v)

        scale = jnp.float32(V)
        upper = jnp.int32(V - 1)
        for j in range(n_chunks):
            for i in range(_IDX_CHUNK // _LANES):
                off = j * _IDX_CHUNK + i * _LANES
                v = ind_v[pl.ds(off, _LANES)]
                v = jnp.minimum(jnp.maximum(v, jnp.float32(0.0)), jnp.float32(1.0))
                row = jnp.minimum((v * scale).astype(jnp.int32), upper)
                idx_v[j, pl.ds(i * _LANES, _LANES)] = row

        copies = [
            pltpu.async_copy(
                items_hbm.at[idx_v.at[j]],
                rows_v.at[pl.ds(j * _IDX_CHUNK, _IDX_CHUNK)],
                sem,
            )
            for j in range(n_chunks)
        ]
        for c in copies:
            c.wait()

        pltpu.sync_copy(rows_v, out_hbm.at[pl.ds(base, b_per_w)])

    return gather_kernel(indices, items)
